# baseline (speedup 1.0000x reference)
import functools

import jax
import jax.numpy as jnp
from jax import lax
from jax.experimental import pallas as pl
from jax.experimental.pallas import tpu as pltpu
from jax.experimental.pallas import tpu_sc as plsc

EPS = 1e-12
LANES = 16
RQ = 4
JU = 6


def _sc_embed_ln(xf, ttf, token_emb, pos_emb, type_emb, ln_gamma, ln_beta,
                 *, n_tokens, seq, hid):
    NC, NS = 2, 16
    NW = NC * NS
    tpw = n_tokens // NW
    CS = 32
    bpw = tpw // seq
    PS = CS // bpw
    n_units = seq // PS
    JD = hid // LANES
    n_types = type_emb.shape[0]

    mesh = plsc.VectorSubcoreMesh(core_axis_name="c", subcore_axis_name="s")

    @functools.partial(
        pl.kernel,
        out_type=jax.ShapeDtypeStruct((n_tokens, hid), jnp.float32),
        mesh=mesh,
        scratch_types=[
            pltpu.VMEM((tpw,), jnp.int32),
            pltpu.VMEM((tpw + LANES,), jnp.int32),
            pltpu.VMEM((CS, hid), jnp.float32),
            pltpu.VMEM((CS, hid), jnp.float32),
            pltpu.VMEM((PS, hid), jnp.float32),
            pltpu.VMEM((PS, hid), jnp.float32),
            pltpu.VMEM((n_types, hid), jnp.float32),
            pltpu.VMEM((hid,), jnp.float32),
            pltpu.VMEM((hid,), jnp.float32),
            pltpu.VMEM((hid,), jnp.float32),
            pltpu.VMEM((CS, LANES), jnp.float32),
            pltpu.VMEM((CS, LANES), jnp.float32),
            pltpu.SemaphoreType.DMA,
            pltpu.SemaphoreType.DMA,
            pltpu.SemaphoreType.DMA,
            pltpu.SemaphoreType.DMA,
            pltpu.SemaphoreType.DMA,
            pltpu.SemaphoreType.DMA,
        ],
    )
    def k(x_h, tt_h, tok_h, pos_h, typ_h, g_h, b_h, out_h,
          idx_all, tt_all, buf0, buf1, pbuf0, pbuf1, typ_v, dt_v, g_v, b_v,
          mean_s, rstd_s, gsem0, gsem1, wsem0, wsem1, psem0, psem1):
        buf = (buf0, buf1)
        pbufs = (pbuf0, pbuf1)
        gsem = (gsem0, gsem1)
        wsem = (wsem0, wsem1)
        psem = (psem0, psem1)

        wid = lax.axis_index("s") * NC + lax.axis_index("c")
        base = wid * tpw
        pltpu.sync_copy(x_h.at[pl.ds(base, tpw)], idx_all)
        pltpu.sync_copy(tt_h.at[pl.ds(base, tpw)], tt_all.at[pl.ds(0, tpw)])
        pltpu.sync_copy(g_h, g_v)
        pltpu.sync_copy(b_h, b_v)
        pltpu.sync_copy(typ_h, typ_v)
        for j in range(JD):
            sl = pl.ds(j * LANES, LANES)
            dt_v[sl] = typ_v[1, sl] - typ_v[0, sl]

        inv_d = jnp.float32(1.0 / hid)
        dnums = lax.GatherDimensionNumbers(
            offset_dims=(), collapsed_slice_dims=(0,), start_index_map=(0,))

        def dyn_gather(v, perm):
            return lax.gather(
                v, perm[:, None], dnums, slice_sizes=(1,),
                mode=lax.GatherScatterMode.PROMISE_IN_BOUNDS)

        def splat(v, lane):
            return dyn_gather(v, lax.broadcast(lane, (LANES,)))

        def lane_sum_multi(vs):
            for sh in (8, 4, 2, 1):
                perm = jnp.arange(LANES, dtype=jnp.int32) ^ jnp.int32(sh)
                gs = [dyn_gather(v, perm) for v in vs]
                vs = [v + g for v, g in zip(vs, gs)]
            return vs

        def fetch(u, p):
            s0 = u * PS
            for b in range(bpw):
                pltpu.async_copy(
                    tok_h.at[idx_all.at[pl.ds(b * seq + s0, PS)]],
                    buf[p].at[pl.ds(b * PS, PS)], gsem[p])

        def wait_gather(p):
            for b in range(bpw):
                pltpu.make_async_copy(
                    tok_h.at[idx_all.at[pl.ds(0, PS)]],
                    buf[p].at[pl.ds(0, PS)], gsem[p]).wait()

        def load_pbuf(u, p):
            pltpu.async_copy(pos_h.at[pl.ds(u * PS, PS)], pbufs[p], psem[p])

        def wait_pbuf(p):
            pltpu.make_async_copy(pos_h.at[pl.ds(0, PS)], pbufs[p],
                                  psem[p]).wait()

        def compute(u, p):
            bp = buf[p]
            pb = pbufs[p]
            s0 = u * PS
            t16s = [tt_all[pl.ds(i * seq + s0, LANES)].astype(jnp.float32)
                    for i in range(bpw)]
            NR = 2 * RQ

            def grp1_body(qq, carry):
                rows = [PS * i + 2 * qq + h
                        for h in range(2) for i in range(bpw)]
                tf = [splat(t16s[i], 2 * qq + h)
                      for h in range(2) for i in range(bpw)]

                zero = jnp.zeros((LANES,), jnp.float32)
                accs = [zero] * NR
                acc2s = [zero] * NR
                for j in range(JD):
                    sl = pl.ds(j * LANES, LANES)
                    t0j = typ_v[0, sl]
                    dtj = dt_v[sl]
                    pq = [pb[2 * qq + h, sl] for h in range(2)]
                    la = [bp[r, sl] for r in rows]
                    mm = [t * dtj for t in tf]
                    s1 = [a + t0j for a in la]
                    s2 = [pq[k // bpw] + m for k, m in enumerate(mm)]
                    vv = [a + b for a, b in zip(s1, s2)]
                    for k in range(NR):
                        bp[rows[k], sl] = vv[k]
                    accs = [a + v for a, v in zip(accs, vv)]
                    sq = [v * v for v in vv]
                    acc2s = [a + s for a, s in zip(acc2s, sq)]

                sums = lane_sum_multi(accs + acc2s)
                ms = [s * inv_d for s in sums[:NR]]
                exs = [s * inv_d - m * m + jnp.float32(EPS)
                       for s, m in zip(sums[NR:], ms)]
                xis = [lax.bitcast_convert_type(ex, jnp.int32) for ex in exs]
                yis = [jnp.int32(0x5F3759DF)
                       - lax.shift_right_arithmetic(xi, jnp.int32(1))
                       for xi in xis]
                ys = [lax.bitcast_convert_type(yi, jnp.float32)
                      for yi in yis]
                for _ in range(3):
                    ts = [jnp.float32(0.5) * ex * y * y
                          for ex, y in zip(exs, ys)]
                    ys = [y * (jnp.float32(1.5) - t)
                          for y, t in zip(ys, ts)]
                for k in range(NR):
                    mean_s[rows[k], :] = ms[k]
                    rstd_s[rows[k], :] = ys[k]
                return carry

            lax.fori_loop(0, PS // 2, grp1_body, 0, unroll=False)

            RO = 2 * RQ

            def oct2_body(q, carry):
                r0 = q * RO
                ms = [mean_s[r0 + i, :] for i in range(RO)]
                ys = [rstd_s[r0 + i, :] for i in range(RO)]
                for j in range(JD):
                    sl = pl.ds(j * LANES, LANES)
                    gj = g_v[sl]
                    bj = b_v[sl]
                    hs = [bp[r0 + i, sl] for i in range(RO)]
                    d0 = [h - m for h, m in zip(hs, ms)]
                    d1 = [d * y for d, y in zip(d0, ys)]
                    d2 = [d * gj for d in d1]
                    d3 = [d + bj for d in d2]
                    for i in range(RO):
                        bp[r0 + i, sl] = d3[i]
                return carry

            lax.fori_loop(0, CS // RO, oct2_body, 0, unroll=False)

        def write(u, p):
            s0 = u * PS
            for b in range(bpw):
                pltpu.async_copy(
                    buf[p].at[pl.ds(b * PS, PS)],
                    out_h.at[pl.ds(base + b * seq + s0, PS)], wsem[p])

        def wait_write(p):
            for b in range(bpw):
                pltpu.make_async_copy(
                    buf[p].at[pl.ds(0, PS)],
                    out_h.at[pl.ds(0, PS)], wsem[p]).wait()

        fetch(jnp.int32(0), 0)
        load_pbuf(jnp.int32(0), 0)

        def pair_body(kk, carry):
            for p in (0, 1):
                u = kk * 2 + p

                @pl.when(u >= 1)
                def _():
                    wait_write(1 - p)

                @pl.when(u < n_units - 1)
                def _():
                    fetch(u + 1, 1 - p)
                    load_pbuf(u + 1, 1 - p)

                wait_pbuf(p)
                wait_gather(p)
                compute(u, p)
                write(u, p)
            return carry

        lax.fori_loop(0, n_units // 2, pair_body, 0, unroll=False)

        wait_write(1)

    return k(xf, ttf, token_emb, pos_emb, type_emb, ln_gamma, ln_beta)


def kernel(x, token_type_ids, token_emb, pos_emb, type_emb, ln_gamma, ln_beta):
    batch, seq = x.shape
    hid = token_emb.shape[1]
    n_tokens = batch * seq
    out = _sc_embed_ln(
        x.reshape(n_tokens), token_type_ids.reshape(n_tokens),
        token_emb, pos_emb, type_emb, ln_gamma, ln_beta,
        n_tokens=n_tokens, seq=seq, hid=hid)
    return out.reshape(batch, seq, hid)


# speedup vs baseline: 2.1521x; 2.1521x over previous
import functools

import jax
import jax.numpy as jnp
from jax import lax
from jax.experimental import pallas as pl
from jax.experimental.pallas import tpu as pltpu
from jax.experimental.pallas import tpu_sc as plsc

EPS = 1e-12
LANES = 16
RQ = 4
JU = 6


def _sc_embed_ln(xf, ttf, token_emb, pos_emb, type_emb, ln_gamma, ln_beta,
                 *, n_tokens, seq, hid):
    NC, NS = 2, 16
    NW = NC * NS
    tpw = n_tokens // NW
    CS = 32
    bpw = tpw // seq
    PS = CS // bpw
    n_units = seq // PS
    JD = hid // LANES
    n_types = type_emb.shape[0]

    mesh = plsc.VectorSubcoreMesh(core_axis_name="c", subcore_axis_name="s")

    @functools.partial(
        pl.kernel,
        out_type=jax.ShapeDtypeStruct((n_tokens, hid), jnp.float32),
        mesh=mesh,
        scratch_types=[
            pltpu.VMEM((tpw,), jnp.int32),
            pltpu.VMEM((tpw + LANES,), jnp.int32),
            pltpu.VMEM((CS, hid), jnp.float32),
            pltpu.VMEM((CS, hid), jnp.float32),
            pltpu.VMEM((PS, hid), jnp.float32),
            pltpu.VMEM((PS, hid), jnp.float32),
            pltpu.VMEM((n_types, hid), jnp.float32),
            pltpu.VMEM((hid,), jnp.float32),
            pltpu.VMEM((hid,), jnp.float32),
            pltpu.VMEM((hid,), jnp.float32),
            pltpu.VMEM((CS, LANES), jnp.float32),
            pltpu.VMEM((CS, LANES), jnp.float32),
            pltpu.SemaphoreType.DMA,
            pltpu.SemaphoreType.DMA,
            pltpu.SemaphoreType.DMA,
            pltpu.SemaphoreType.DMA,
            pltpu.SemaphoreType.DMA,
            pltpu.SemaphoreType.DMA,
        ],
    )
    def k(x_h, tt_h, tok_h, pos_h, typ_h, g_h, b_h, out_h,
          idx_all, tt_all, buf0, buf1, pbuf0, pbuf1, typ_v, dt_v, g_v, b_v,
          mean_s, rstd_s, gsem0, gsem1, wsem0, wsem1, psem0, psem1):
        buf = (buf0, buf1)
        pbufs = (pbuf0, pbuf1)
        gsem = (gsem0, gsem1)
        wsem = (wsem0, wsem1)
        psem = (psem0, psem1)

        wid = lax.axis_index("s") * NC + lax.axis_index("c")
        base = wid * tpw
        pltpu.sync_copy(x_h.at[pl.ds(base, tpw)], idx_all)
        pltpu.sync_copy(tt_h.at[pl.ds(base, tpw)], tt_all.at[pl.ds(0, tpw)])
        pltpu.sync_copy(g_h, g_v)
        pltpu.sync_copy(b_h, b_v)
        pltpu.sync_copy(typ_h, typ_v)
        for j in range(JD):
            sl = pl.ds(j * LANES, LANES)
            dt_v[sl] = typ_v[1, sl] - typ_v[0, sl]

        inv_d = jnp.float32(1.0 / hid)
        dnums = lax.GatherDimensionNumbers(
            offset_dims=(), collapsed_slice_dims=(0,), start_index_map=(0,))

        def dyn_gather(v, perm):
            return lax.gather(
                v, perm[:, None], dnums, slice_sizes=(1,),
                mode=lax.GatherScatterMode.PROMISE_IN_BOUNDS)

        def splat(v, lane):
            return dyn_gather(v, lax.broadcast(lane, (LANES,)))

        def lane_sum_multi(vs):
            for sh in (8, 4, 2, 1):
                perm = jnp.arange(LANES, dtype=jnp.int32) ^ jnp.int32(sh)
                gs = [dyn_gather(v, perm) for v in vs]
                vs = [v + g for v, g in zip(vs, gs)]
            return vs

        def fetch(u, p):
            s0 = u * PS
            for b in range(bpw):
                pltpu.async_copy(
                    tok_h.at[idx_all.at[pl.ds(b * seq + s0, PS)]],
                    buf[p].at[pl.ds(b * PS, PS)], gsem[p])

        def wait_gather(p):
            for b in range(bpw):
                pltpu.make_async_copy(
                    tok_h.at[idx_all.at[pl.ds(0, PS)]],
                    buf[p].at[pl.ds(0, PS)], gsem[p]).wait()

        def load_pbuf(u, p):
            pltpu.async_copy(pos_h.at[pl.ds(u * PS, PS)], pbufs[p], psem[p])

        def wait_pbuf(p):
            pltpu.make_async_copy(pos_h.at[pl.ds(0, PS)], pbufs[p],
                                  psem[p]).wait()

        def compute(u, p):
            bp = buf[p]
            pbuf = pbufs[p]
            s0 = u * PS
            t16s = [tt_all[pl.ds(i * seq + s0, LANES)].astype(jnp.float32)
                    for i in range(RQ)]

            def quad1_body(q, carry):
                rows = [PS * i + q for i in range(RQ)]
                tf = [splat(t16s[i], q) for i in range(RQ)]

                zero = jnp.zeros((LANES,), jnp.float32)
                accs = [zero] * RQ
                acc2s = [zero] * RQ
                for j in range(JD):
                    sl = pl.ds(j * LANES, LANES)
                    t0j = typ_v[0, sl]
                    dtj = dt_v[sl]
                    pq = pbuf[q, sl]
                    la = [bp[r, sl] for r in rows]
                    mm = [tf[i] * dtj for i in range(RQ)]
                    s1 = [a + t0j for a in la]
                    s2 = [pq + m for m in mm]
                    vv = [a + b for a, b in zip(s1, s2)]
                    for i in range(RQ):
                        bp[rows[i], sl] = vv[i]
                    accs = [a + v for a, v in zip(accs, vv)]
                    sq = [v * v for v in vv]
                    acc2s = [a + s for a, s in zip(acc2s, sq)]

                sums = lane_sum_multi(accs + acc2s)
                ms = [s * inv_d for s in sums[:RQ]]
                exs = [s * inv_d - m * m + jnp.float32(EPS)
                       for s, m in zip(sums[RQ:], ms)]
                xis = [lax.bitcast_convert_type(ex, jnp.int32) for ex in exs]
                yis = [jnp.int32(0x5F3759DF)
                       - lax.shift_right_arithmetic(xi, jnp.int32(1))
                       for xi in xis]
                ys = [lax.bitcast_convert_type(yi, jnp.float32)
                      for yi in yis]
                for _ in range(3):
                    ts = [jnp.float32(0.5) * ex * y * y
                          for ex, y in zip(exs, ys)]
                    ys = [y * (jnp.float32(1.5) - t)
                          for y, t in zip(ys, ts)]
                for i in range(RQ):
                    mean_s[rows[i], :] = ms[i]
                    rstd_s[rows[i], :] = ys[i]
                return carry

            lax.fori_loop(0, PS, quad1_body, 0, unroll=False)

            RO = 2 * RQ

            def oct2_body(q, carry):
                r0 = q * RO
                ms = [mean_s[r0 + i, :] for i in range(RO)]
                ys = [rstd_s[r0 + i, :] for i in range(RO)]
                for j in range(JD):
                    sl = pl.ds(j * LANES, LANES)
                    gj = g_v[sl]
                    bj = b_v[sl]
                    hs = [bp[r0 + i, sl] for i in range(RO)]
                    d0 = [h - m for h, m in zip(hs, ms)]
                    d1 = [d * y for d, y in zip(d0, ys)]
                    d2 = [d * gj for d in d1]
                    d3 = [d + bj for d in d2]
                    for i in range(RO):
                        bp[r0 + i, sl] = d3[i]
                return carry

            lax.fori_loop(0, CS // RO, oct2_body, 0, unroll=False)

        def write(u, p):
            s0 = u * PS
            for b in range(bpw):
                pltpu.async_copy(
                    buf[p].at[pl.ds(b * PS, PS)],
                    out_h.at[pl.ds(base + b * seq + s0, PS)], wsem[p])

        def wait_write(p):
            for b in range(bpw):
                pltpu.make_async_copy(
                    buf[p].at[pl.ds(0, PS)],
                    out_h.at[pl.ds(0, PS)], wsem[p]).wait()

        fetch(jnp.int32(0), 0)
        load_pbuf(jnp.int32(0), 0)

        def pair_body(kk, carry):
            for p in (0, 1):
                u = kk * 2 + p

                @pl.when(u >= 1)
                def _():
                    wait_write(1 - p)

                @pl.when(u < n_units - 1)
                def _():
                    fetch(u + 1, 1 - p)
                    load_pbuf(u + 1, 1 - p)

                wait_pbuf(p)
                wait_gather(p)
                compute(u, p)
                write(u, p)
            return carry

        lax.fori_loop(0, n_units // 2, pair_body, 0, unroll=False)

        wait_write(1)

    return k(xf, ttf, token_emb, pos_emb, type_emb, ln_gamma, ln_beta)


def kernel(x, token_type_ids, token_emb, pos_emb, type_emb, ln_gamma, ln_beta):
    batch, seq = x.shape
    hid = token_emb.shape[1]
    n_tokens = batch * seq
    out = _sc_embed_ln(
        x.reshape(n_tokens), token_type_ids.reshape(n_tokens),
        token_emb, pos_emb, type_emb, ln_gamma, ln_beta,
        n_tokens=n_tokens, seq=seq, hid=hid)
    return out.reshape(batch, seq, hid)
